# Initial kernel scaffold; baseline (speedup 1.0000x reference)
import jax
import jax.numpy as jnp
from jax.experimental import pallas as pl


def kernel(input_ids, table):
    raise NotImplementedError("write your pallas kernel here")


# speedup vs baseline: 1.5411x; 1.5411x over previous
import functools

import jax
import jax.numpy as jnp
from jax import lax
from jax.experimental import pallas as pl
from jax.experimental.pallas import tpu as pltpu
from jax.experimental.pallas import tpu_sc as plsc

HIDDEN = 1024
BATCH = 4
SEQ = 4096
TOTAL = BATCH * SEQ

NC = 2
NS = 16
NW = NC * NS
B_PER_W = TOTAL // NW
CHUNK = 64
N_CHUNKS = B_PER_W // CHUNK

_mesh = plsc.VectorSubcoreMesh(core_axis_name="c", subcore_axis_name="s")


@functools.partial(
    pl.kernel,
    mesh=_mesh,
    out_type=jax.ShapeDtypeStruct((TOTAL, HIDDEN), jnp.float32),
    scratch_types=[
        pltpu.VMEM((N_CHUNKS, CHUNK), jnp.int32),
        pltpu.VMEM((CHUNK, HIDDEN), jnp.float32),
        pltpu.SemaphoreType.DMA,
    ],
)
def _embed_kernel(table_hbm, idx_hbm, out_hbm, idx_v, rows_v, gsem):
    wid = lax.axis_index("s") * NC + lax.axis_index("c")
    base = wid * B_PER_W
    pltpu.sync_copy(idx_hbm.at[wid], idx_v)
    for c in range(N_CHUNKS):
        pltpu.async_copy(table_hbm.at[idx_v.at[c]], rows_v, gsem).wait()
        pltpu.sync_copy(rows_v, out_hbm.at[pl.ds(base + c * CHUNK, CHUNK)])


def kernel(input_ids, table):
    ids = input_ids.reshape(NW, N_CHUNKS, CHUNK).astype(jnp.int32)
    out = _embed_kernel(table, ids)
    return out.reshape(BATCH, SEQ, HIDDEN)


# speedup vs baseline: 1.5753x; 1.0222x over previous
import functools

import jax
import jax.numpy as jnp
from jax import lax
from jax.experimental import pallas as pl
from jax.experimental.pallas import tpu as pltpu
from jax.experimental.pallas import tpu_sc as plsc

HIDDEN = 1024
BATCH = 4
SEQ = 4096
TOTAL = BATCH * SEQ

NC = 2
NS = 16
NW = NC * NS
B_PER_W = TOTAL // NW
CHUNK = 32
N_CHUNKS = B_PER_W // CHUNK
NBUF = 3

_mesh = plsc.VectorSubcoreMesh(core_axis_name="c", subcore_axis_name="s")


@functools.partial(
    pl.kernel,
    mesh=_mesh,
    out_type=jax.ShapeDtypeStruct((TOTAL, HIDDEN), jnp.float32),
    scratch_types=[
        pltpu.VMEM((N_CHUNKS, CHUNK), jnp.int32),
        pltpu.VMEM((NBUF, CHUNK, HIDDEN), jnp.float32),
        pltpu.SemaphoreType.DMA((NBUF,)),
        pltpu.SemaphoreType.DMA((NBUF,)),
    ],
)
def _embed_kernel(table_hbm, idx_hbm, out_hbm, idx_v, rows_v, gsem, osem):
    wid = lax.axis_index("s") * NC + lax.axis_index("c")
    base = wid * B_PER_W
    pltpu.sync_copy(idx_hbm.at[wid], idx_v)

    def gather(c, buf):
        pltpu.async_copy(table_hbm.at[idx_v.at[c]], rows_v.at[buf], gsem.at[buf])

    def wait_gather(c, buf):
        pltpu.make_async_copy(
            table_hbm.at[idx_v.at[c]], rows_v.at[buf], gsem.at[buf]
        ).wait()

    def write(c, buf):
        return pltpu.async_copy(
            rows_v.at[buf], out_hbm.at[pl.ds(base + c * CHUNK, CHUNK)], osem.at[buf]
        )

    gather(0, 0)
    writes = [None] * NBUF
    for t in range(N_CHUNKS):
        buf = t % NBUF
        wait_gather(t, buf)
        writes[buf] = write(t, buf)
        m = t + 1
        if m < N_CHUNKS:
            mbuf = m % NBUF
            if writes[mbuf] is not None:
                writes[mbuf].wait()
            gather(m, mbuf)
    writes[(N_CHUNKS - 1) % NBUF].wait()


def kernel(input_ids, table):
    ids = input_ids.reshape(NW, N_CHUNKS, CHUNK).astype(jnp.int32)
    out = _embed_kernel(table, ids)
    return out.reshape(BATCH, SEQ, HIDDEN)


# speedup vs baseline: 1.5862x; 1.0069x over previous
import functools

import jax
import jax.numpy as jnp
from jax import lax
from jax.experimental import pallas as pl
from jax.experimental.pallas import tpu as pltpu
from jax.experimental.pallas import tpu_sc as plsc

HIDDEN = 1024
BATCH = 4
SEQ = 4096
TOTAL = BATCH * SEQ

NC = 2
NS = 16
NW = NC * NS
B_PER_W = TOTAL // NW
W_PER_SEQ = SEQ // B_PER_W
CHUNK = 32
N_CHUNKS = B_PER_W // CHUNK
NBUF = 3

_mesh = plsc.VectorSubcoreMesh(core_axis_name="c", subcore_axis_name="s")


@functools.partial(
    pl.kernel,
    mesh=_mesh,
    out_type=jax.ShapeDtypeStruct((BATCH, SEQ, HIDDEN), jnp.float32),
    scratch_types=[
        pltpu.VMEM((B_PER_W,), jnp.int32),
        pltpu.VMEM((NBUF, CHUNK, HIDDEN), jnp.float32),
        pltpu.SemaphoreType.DMA((NBUF,)),
        pltpu.SemaphoreType.DMA((NBUF,)),
    ],
)
def _embed_kernel(idx_hbm, table_hbm, out_hbm, idx_v, rows_v, gsem, osem):
    wid = lax.axis_index("s") * NC + lax.axis_index("c")
    b = wid // W_PER_SEQ
    scol = (wid % W_PER_SEQ) * B_PER_W
    pltpu.sync_copy(idx_hbm.at[b, pl.ds(scol, B_PER_W)], idx_v)

    def gather(c, buf):
        pltpu.async_copy(
            table_hbm.at[idx_v.at[pl.ds(c * CHUNK, CHUNK)]],
            rows_v.at[buf],
            gsem.at[buf],
        )

    def wait_gather(c, buf):
        pltpu.make_async_copy(
            table_hbm.at[idx_v.at[pl.ds(c * CHUNK, CHUNK)]],
            rows_v.at[buf],
            gsem.at[buf],
        ).wait()

    def write(c, buf):
        return pltpu.async_copy(
            rows_v.at[buf],
            out_hbm.at[b, pl.ds(scol + c * CHUNK, CHUNK)],
            osem.at[buf],
        )

    gather(0, 0)
    writes = [None] * NBUF
    for t in range(N_CHUNKS):
        buf = t % NBUF
        wait_gather(t, buf)
        writes[buf] = write(t, buf)
        m = t + 1
        if m < N_CHUNKS:
            mbuf = m % NBUF
            if writes[mbuf] is not None:
                writes[mbuf].wait()
            gather(m, mbuf)
    writes[(N_CHUNKS - 1) % NBUF].wait()


def kernel(input_ids, table):
    return _embed_kernel(input_ids.astype(jnp.int32), table)


# speedup vs baseline: 1.6462x; 1.0378x over previous
import functools

import jax
import jax.numpy as jnp
from jax import lax
from jax.experimental import pallas as pl
from jax.experimental.pallas import tpu as pltpu
from jax.experimental.pallas import tpu_sc as plsc

HIDDEN = 1024
BATCH = 4
SEQ = 4096
TOTAL = BATCH * SEQ

NC = 2
NS = 16
NW = NC * NS
B_PER_W = TOTAL // NW
W_PER_SEQ = SEQ // B_PER_W
CHUNK = 16
N_CHUNKS = B_PER_W // CHUNK
NBUF = 6
LEAD = 3

_mesh = plsc.VectorSubcoreMesh(core_axis_name="c", subcore_axis_name="s")


@functools.partial(
    pl.kernel,
    mesh=_mesh,
    out_type=jax.ShapeDtypeStruct((BATCH, SEQ, HIDDEN), jnp.float32),
    scratch_types=[
        pltpu.VMEM((B_PER_W,), jnp.int32),
        pltpu.VMEM((NBUF, CHUNK, HIDDEN), jnp.float32),
        pltpu.SemaphoreType.DMA((NBUF,)),
        pltpu.SemaphoreType.DMA((NBUF,)),
    ],
)
def _embed_kernel(idx_hbm, table_hbm, out_hbm, idx_v, rows_v, gsem, osem):
    wid = lax.axis_index("s") * NC + lax.axis_index("c")
    b = wid // W_PER_SEQ
    scol = (wid % W_PER_SEQ) * B_PER_W
    pltpu.sync_copy(idx_hbm.at[b, pl.ds(scol, B_PER_W)], idx_v)

    def gather(c, buf):
        pltpu.async_copy(
            table_hbm.at[idx_v.at[pl.ds(c * CHUNK, CHUNK)]],
            rows_v.at[buf],
            gsem.at[buf],
        )

    def wait_gather(c, buf):
        pltpu.make_async_copy(
            table_hbm.at[idx_v.at[pl.ds(c * CHUNK, CHUNK)]],
            rows_v.at[buf],
            gsem.at[buf],
        ).wait()

    def write(c, buf):
        return pltpu.async_copy(
            rows_v.at[buf],
            out_hbm.at[b, pl.ds(scol + c * CHUNK, CHUNK)],
            osem.at[buf],
        )

    for p in range(LEAD):
        gather(p, p)
    writes = [None] * NBUF
    for t in range(N_CHUNKS):
        buf = t % NBUF
        wait_gather(t, buf)
        writes[buf] = write(t, buf)
        m = t + LEAD
        if m < N_CHUNKS:
            mbuf = m % NBUF
            if writes[mbuf] is not None:
                writes[mbuf].wait()
            gather(m, mbuf)
    for w in writes:
        if w is not None:
            w.wait()


def kernel(input_ids, table):
    return _embed_kernel(input_ids.astype(jnp.int32), table)
